# baseline (speedup 1.0000x reference)
import functools

import jax
import jax.numpy as jnp
from jax import lax
from jax.experimental import pallas as pl
from jax.experimental.pallas import tpu as pltpu
from jax.experimental.pallas import tpu_sc as plsc

HIDDEN = 64
L = 16
NC, NS = 2, 16
NW = NC * NS
CHUNK = 64
NBUF = 2


@functools.partial(jax.jit, static_argnames=("B",))
def _run(user, item, user_factors, item_factors, user_biases, item_biases, B):
    b_per_w = B // NW
    n_chunks = b_per_w // CHUNK
    mesh = plsc.VectorSubcoreMesh(core_axis_name="c", subcore_axis_name="s")

    @functools.partial(
        pl.kernel,
        mesh=mesh,
        compiler_params=pltpu.CompilerParams(
            needs_layout_passes=False, use_tc_tiling_on_sc=False,
            skip_device_barrier=True, disable_bounds_checks=True,
            disable_semaphore_checks=True),
        out_type=jax.ShapeDtypeStruct((B,), jnp.float32),
        scratch_types=[
            pltpu.VMEM((NBUF, CHUNK), jnp.int32),
            pltpu.VMEM((NBUF, CHUNK), jnp.int32),
            pltpu.VMEM((NBUF, CHUNK, HIDDEN), jnp.float32),
            pltpu.VMEM((NBUF, CHUNK, HIDDEN), jnp.float32),
            pltpu.VMEM((NBUF, CHUNK, HIDDEN), jnp.float32),
            pltpu.VMEM((NBUF, CHUNK, HIDDEN), jnp.float32),
            pltpu.VMEM((NBUF, CHUNK), jnp.float32),
            pltpu.SemaphoreType.DMA,
            pltpu.SemaphoreType.DMA,
        ],
    )
    def k(user_hbm, item_hbm, uf_hbm, if_hbm, ub_hbm, ib_hbm, out_hbm,
          uidx_v, iidx_v, uf_v, if_v, ub_v, ib_v, o_v, semA, semB):
        wid = lax.axis_index("s") * NC + lax.axis_index("c")
        base = wid * b_per_w
        lane = lax.iota(jnp.int32, L)
        sems = (semA, semB)

        def fire(c, b, sem):
            off = base + c * CHUNK
            pltpu.sync_copy(user_hbm.at[pl.ds(off, CHUNK)], uidx_v.at[b])
            pltpu.sync_copy(item_hbm.at[pl.ds(off, CHUNK)], iidx_v.at[b])
            pltpu.async_copy(uf_hbm.at[uidx_v.at[b]], uf_v.at[b], sem)
            pltpu.async_copy(if_hbm.at[iidx_v.at[b]], if_v.at[b], sem)
            pltpu.async_copy(ub_hbm.at[uidx_v.at[b]], ub_v.at[b], sem)
            pltpu.async_copy(ib_hbm.at[iidx_v.at[b]], ib_v.at[b], sem)

        def drain(b, sem):
            pltpu.make_async_copy(uf_hbm.at[uidx_v.at[b]], uf_v.at[b], sem).wait()
            pltpu.make_async_copy(if_hbm.at[iidx_v.at[b]], if_v.at[b], sem).wait()
            pltpu.make_async_copy(ub_hbm.at[uidx_v.at[b]], ub_v.at[b], sem).wait()
            pltpu.make_async_copy(ib_hbm.at[iidx_v.at[b]], ib_v.at[b], sem).wait()

        def compute(c, b):
            off = base + c * CHUNK
            for g in range(CHUNK // L):
                vec = jnp.zeros((L,), jnp.float32)
                for jj in range(L):
                    j = g * L + jj
                    acc = jnp.zeros((L,), jnp.float32)
                    for kk in range(HIDDEN // L):
                        sl = pl.ds(kk * L, L)
                        acc = acc + (uf_v[b, j, sl] * if_v[b, j, sl]
                                     + ub_v[b, j, sl] + ib_v[b, j, sl])
                    vec = jnp.where(lane == jj, jnp.sum(acc), vec)
                o_v[b, pl.ds(g * L, L)] = vec
            pltpu.sync_copy(o_v.at[b], out_hbm.at[pl.ds(off, CHUNK)])

        fire(0, 0, semA)

        def pair_body(p, _):
            g = p * NBUF
            fire(g + 1, 1, semB)
            drain(0, semA)
            compute(g, 0)

            @pl.when(g + 2 < n_chunks)
            def _():
                fire(g + 2, 0, semA)

            drain(1, semB)
            compute(g + 1, 1)
            return 0

        lax.fori_loop(0, n_chunks // NBUF, pair_body, 0)

    return k(user.astype(jnp.int32), item.astype(jnp.int32),
             user_factors, item_factors, user_biases, item_biases)


def kernel(user, item, user_factors, item_factors, user_biases, item_biases):
    B = user.shape[0]
    out = _run(user, item, user_factors, item_factors, user_biases,
               item_biases, B)
    return out.reshape(B, 1)
